# baseline (speedup 1.0000x reference)
import jax
import jax.numpy as jnp
from jax.experimental import pallas as pl

VOCAB = 1000
ROWS = 1024
SEQ = 50
TOKENS = ROWS * SEQ
BLOCK_T = 1024
GRID = TOKENS // BLOCK_T


def _onehot_body(ids_ref, mask_ref, out_ref):
    ids = ids_ref[0]
    mask = mask_ref[0]
    iota = jax.lax.broadcasted_iota(jnp.int32, (BLOCK_T, VOCAB), 1)
    out_ref[...] = jnp.where(iota == ids, mask, 0.0)


def kernel(input_ids, attention_mask):
    ids = input_ids.astype(jnp.int32).reshape(GRID, BLOCK_T, 1)
    mask = attention_mask.astype(jnp.float32).reshape(GRID, BLOCK_T, 1)
    out = pl.pallas_call(
        _onehot_body,
        grid=(GRID,),
        in_specs=[
            pl.BlockSpec((1, BLOCK_T, 1), lambda i: (i, 0, 0)),
            pl.BlockSpec((1, BLOCK_T, 1), lambda i: (i, 0, 0)),
        ],
        out_specs=pl.BlockSpec((BLOCK_T, VOCAB), lambda i: (i, 0)),
        out_shape=jax.ShapeDtypeStruct((TOKENS, VOCAB), jnp.float32),
    )(ids, mask)
    return out.reshape(ROWS, SEQ, VOCAB)


# speedup vs baseline: 6.7350x; 6.7350x over previous
import jax
import jax.numpy as jnp
from jax.experimental import pallas as pl

VOCAB = 1000
ROWS = 1024
SEQ = 50


def _onehot_body(ids_ref, mask_ref, out_ref):
    ids = ids_ref[0]
    mask = mask_ref[0]
    iota_v = jax.lax.broadcasted_iota(jnp.int32, (VOCAB, ROWS), 0)
    out_ref[0] = jnp.where(iota_v == ids, mask, 0.0)


def kernel(input_ids, attention_mask):
    ids_t = input_ids.astype(jnp.int32).T.reshape(SEQ, 1, ROWS)
    mask_t = attention_mask.astype(jnp.float32).T.reshape(SEQ, 1, ROWS)
    out_t = pl.pallas_call(
        _onehot_body,
        grid=(SEQ,),
        in_specs=[
            pl.BlockSpec((1, 1, ROWS), lambda j: (j, 0, 0)),
            pl.BlockSpec((1, 1, ROWS), lambda j: (j, 0, 0)),
        ],
        out_specs=pl.BlockSpec((1, VOCAB, ROWS), lambda j: (j, 0, 0)),
        out_shape=jax.ShapeDtypeStruct((SEQ, VOCAB, ROWS), jnp.float32),
    )(ids_t, mask_t)
    return jnp.transpose(out_t, (2, 0, 1))
